# Initial kernel scaffold; baseline (speedup 1.0000x reference)
import jax
import jax.numpy as jnp
from jax.experimental import pallas as pl


def kernel(logits, raw_descriptors):
    raise NotImplementedError("write your pallas kernel here")


# speedup vs baseline: 1.0123x; 1.0123x over previous
import functools

import jax
import jax.numpy as jnp
from jax import lax
from jax.experimental import pallas as pl
from jax.experimental.pallas import tpu as pltpu
from jax.experimental.pallas import tpu_sc as plsc

_B, _D, _H, _W = 2, 128, 512, 512
_K = 100
_THR = 0.8
_BORD = 4
_HW = _H * _W
_PAD_SLOTS = 256
_IDX_PAD = 264


def _shift_max_1d(p, axis, dist):
    n = p.shape[axis]
    if axis == 1:
        zero = jnp.zeros((p.shape[0], dist), p.dtype)
        left = jnp.concatenate([p[:, dist:], zero], axis=1)
        right = jnp.concatenate([zero, p[:, : n - dist]], axis=1)
    else:
        zero = jnp.zeros((dist, p.shape[1]), p.dtype)
        left = jnp.concatenate([p[dist:, :], zero], axis=0)
        right = jnp.concatenate([zero, p[: n - dist, :]], axis=0)
    return jnp.maximum(jnp.maximum(left, right), p)


def _nms_topk_body(prob_ref, vals_ref, idx_ref, masked_ref):
    p = prob_ref[0]

    m = _shift_max_1d(p, 1, 1)
    m = _shift_max_1d(m, 1, 3)
    m = _shift_max_1d(m, 0, 1)
    m = _shift_max_1d(m, 0, 3)

    ys = lax.broadcasted_iota(jnp.int32, (_H, _W), 0)
    xs = lax.broadcasted_iota(jnp.int32, (_H, _W), 1)
    border = (ys >= _BORD) & (ys < _H - _BORD) & (xs >= _BORD) & (xs < _W - _BORD)
    mask = (p >= m) & (p > _THR) & border
    masked = jnp.where(mask, p, 0.0)
    masked_ref[...] = masked

    rmax0 = jnp.max(masked.reshape(_H // 8, 8, _W), axis=2)
    rowid = (lax.broadcasted_iota(jnp.int32, (_H // 8, 8), 0) * 8
             + lax.broadcasted_iota(jnp.int32, (_H // 8, 8), 1))
    colid = lax.broadcasted_iota(jnp.int32, (1, _W), 1)
    big = jnp.int32(1 << 20)

    def step(k, rmax):
        mval = jnp.max(rmax)
        rid = jnp.min(jnp.where(rmax == mval, rowid, big))
        row = masked_ref[pl.ds(rid, 1), :]
        col = jnp.min(jnp.where(row == mval, colid, big))
        vals_ref[0, 0, k] = mval
        idx_ref[0, 0, k] = rid * _W + col
        newrow = jnp.where(colid == col, -1.0, row)
        masked_ref[pl.ds(rid, 1), :] = newrow
        return jnp.where(rowid == rid, jnp.max(newrow), rmax)

    lax.fori_loop(0, _K, step, rmax0)


def _nms_topk(prob):
    return pl.pallas_call(
        _nms_topk_body,
        grid=(_B,),
        in_specs=[pl.BlockSpec((1, _H, _W), lambda b: (b, 0, 0))],
        out_specs=[
            pl.BlockSpec((1, 1, _K), lambda b: (b, 0, 0), memory_space=pltpu.SMEM),
            pl.BlockSpec((1, 1, _K), lambda b: (b, 0, 0), memory_space=pltpu.SMEM),
        ],
        out_shape=[
            jax.ShapeDtypeStruct((_B, 1, _K), jnp.float32),
            jax.ShapeDtypeStruct((_B, 1, _K), jnp.int32),
        ],
        scratch_shapes=[pltpu.VMEM((_H, _W), jnp.float32)],
    )(prob)


def _sc_gather_body(desc_hbm, idx_hbm, out_hbm, ibuf, rbuf, obuf, tmp, sem):
    wid = lax.axis_index("s") * 2 + lax.axis_index("c")
    pltpu.sync_copy(idx_hbm.at[pl.ds(wid * 8, 8)], ibuf)
    copies = []
    for j in range(8):
        copies.append(pltpu.async_copy(desc_hbm.at[ibuf.at[j]], rbuf.at[j], sem))
    for cp in copies:
        cp.wait()

    for j in range(8):
        vecs = [rbuf[j, pl.ds(kb * 16, 16)] for kb in range(8)]
        acc = vecs[0] * vecs[0]
        for kb in range(1, 8):
            acc = acc + vecs[kb] * vecs[kb]
        s_v = acc
        for off in (8, 4, 2, 1):
            tmp[pl.ds(0, 16)] = s_v
            tmp[pl.ds(16, 16)] = s_v
            s_v = s_v + tmp[pl.ds(off, 16)]
        bits = plsc.bitcast(s_v, jnp.int32)
        r = plsc.bitcast(jnp.int32(0x5F3759DF) - (bits >> 1), jnp.float32)
        for _ in range(3):
            r = r * (1.5 - 0.5 * s_v * r * r)
        denom = jnp.maximum(s_v * r, 1e-12)
        for kb in range(8):
            obuf[j, pl.ds(kb * 16, 16)] = vecs[kb] / denom

    pltpu.sync_copy(obuf, out_hbm.at[pl.ds(wid * 8, 8)])


@functools.cache
def _make_sc_gather():
    return pl.kernel(
        _sc_gather_body,
        out_type=jax.ShapeDtypeStruct((_PAD_SLOTS, _D), jnp.float32),
        mesh=plsc.VectorSubcoreMesh(core_axis_name="c", subcore_axis_name="s"),
        compiler_params=pltpu.CompilerParams(needs_layout_passes=False),
        scratch_types=[
            pltpu.VMEM((8, _D), jnp.int32),
            pltpu.VMEM((8, _D), jnp.float32),
            pltpu.VMEM((8, _D), jnp.float32),
            pltpu.VMEM((32,), jnp.float32),
            pltpu.SemaphoreType.DMA,
        ],
    )


def kernel(logits, raw_descriptors):
    prob = jax.nn.sigmoid(logits[:, 0])
    vals, idx = _nms_topk(prob)
    vals, idx = vals[:, 0], idx[:, 0]
    y = (idx // _W).astype(jnp.float32)
    x = (idx % _W).astype(jnp.float32)
    positions = jnp.stack([y, x, vals], axis=-1)

    base = idx + jnp.array([[0], [_D * _HW]], jnp.int32)
    base = jnp.pad(base.reshape(-1), (0, _PAD_SLOTS - _B * _K))
    word_idx = base[:, None] + jnp.arange(_D, dtype=jnp.int32)[None, :] * _HW
    desc_flat = raw_descriptors.reshape(-1)
    gathered = _make_sc_gather()(desc_flat, word_idx)
    sparse_desc = gathered[: _B * _K].reshape(_B, _K, _D)
    return positions, sparse_desc


# speedup vs baseline: 1.0869x; 1.0737x over previous
import functools

import jax
import jax.numpy as jnp
from jax import lax
from jax.experimental import pallas as pl
from jax.experimental.pallas import tpu as pltpu
from jax.experimental.pallas import tpu_sc as plsc

_B, _D, _H, _W = 2, 128, 512, 512
_K = 100
_THR = 0.8
_BORD = 4
_HW = _H * _W
_PAD_SLOTS = 256
_IDX_PAD = 264


def _shift_max_1d(p, axis, dist):
    n = p.shape[axis]
    if axis == 1:
        zero = jnp.zeros((p.shape[0], dist), p.dtype)
        left = jnp.concatenate([p[:, dist:], zero], axis=1)
        right = jnp.concatenate([zero, p[:, : n - dist]], axis=1)
    else:
        zero = jnp.zeros((dist, p.shape[1]), p.dtype)
        left = jnp.concatenate([p[dist:, :], zero], axis=0)
        right = jnp.concatenate([zero, p[: n - dist, :]], axis=0)
    return jnp.maximum(jnp.maximum(left, right), p)


def _nms_topk_body(prob_ref, vals_ref, idx_ref, masked_ref):
    p = prob_ref[0]

    m = _shift_max_1d(p, 1, 1)
    m = _shift_max_1d(m, 1, 3)
    m = _shift_max_1d(m, 0, 1)
    m = _shift_max_1d(m, 0, 3)

    ys = lax.broadcasted_iota(jnp.int32, (_H, _W), 0)
    xs = lax.broadcasted_iota(jnp.int32, (_H, _W), 1)
    border = (ys >= _BORD) & (ys < _H - _BORD) & (xs >= _BORD) & (xs < _W - _BORD)
    mask = (p >= m) & (p > _THR) & border
    masked = jnp.where(mask, p, 0.0)
    masked_ref[...] = masked

    rmax0 = jnp.max(masked.reshape(_H // 8, 8, _W), axis=2)
    rowid = (lax.broadcasted_iota(jnp.int32, (_H // 8, 8), 0) * 8
             + lax.broadcasted_iota(jnp.int32, (_H // 8, 8), 1))
    colid = lax.broadcasted_iota(jnp.int32, (1, _W), 1)
    big = jnp.int32(1 << 20)

    def step(k, rmax):
        mval = jnp.max(rmax)
        rid = jnp.min(jnp.where(rmax == mval, rowid, big))
        row = masked_ref[pl.ds(rid, 1), :]
        col = jnp.min(jnp.where(row == mval, colid, big))
        vals_ref[0, 0, k] = mval
        idx_ref[0, 0, k] = rid * _W + col
        newrow = jnp.where(colid == col, -1.0, row)
        masked_ref[pl.ds(rid, 1), :] = newrow
        return jnp.where(rowid == rid, jnp.max(newrow), rmax)

    lax.fori_loop(0, _K, step, rmax0)


def _nms_topk(prob):
    return pl.pallas_call(
        _nms_topk_body,
        grid=(_B,),
        in_specs=[pl.BlockSpec((1, _H, _W), lambda b: (b, 0, 0))],
        out_specs=[
            pl.BlockSpec((1, 1, _K), lambda b: (b, 0, 0), memory_space=pltpu.SMEM),
            pl.BlockSpec((1, 1, _K), lambda b: (b, 0, 0), memory_space=pltpu.SMEM),
        ],
        out_shape=[
            jax.ShapeDtypeStruct((_B, 1, _K), jnp.float32),
            jax.ShapeDtypeStruct((_B, 1, _K), jnp.int32),
        ],
        scratch_shapes=[pltpu.VMEM((_H, _W), jnp.float32)],
    )(prob)


def _sc_gather_body(desc_hbm, idx_hbm, xcol_hbm, out_hbm, ibuf, xwin, rbuf,
                    obuf, tmp, sem):
    wid = lax.axis_index("s") * 2 + lax.axis_index("c")
    pltpu.sync_copy(idx_hbm.at[pl.ds(wid * 8, 8)], ibuf)
    pltpu.sync_copy(xcol_hbm.at[pl.ds(wid * 8, 8)], xwin)
    table = desc_hbm.reshape(_B * _D * _H, _W)

    lanes16 = lax.broadcasted_iota(jnp.int32, (16,), 0)
    for j in range(8):
        pltpu.async_copy(table.at[ibuf.at[j]], rbuf, sem).wait()
        xv = xwin[j]
        vecs = [
            plsc.load_gather(rbuf, [kb * 16 + lanes16, xv])
            for kb in range(8)
        ]
        acc = vecs[0] * vecs[0]
        for kb in range(1, 8):
            acc = acc + vecs[kb] * vecs[kb]
        s_v = acc
        for off in (8, 4, 2, 1):
            tmp[pl.ds(0, 16)] = s_v
            tmp[pl.ds(16, 16)] = s_v
            s_v = s_v + tmp[pl.ds(off, 16)]
        bits = plsc.bitcast(s_v, jnp.int32)
        r = plsc.bitcast(jnp.int32(0x5F3759DF) - (bits >> 1), jnp.float32)
        for _ in range(3):
            r = r * (1.5 - 0.5 * s_v * r * r)
        denom = jnp.maximum(s_v * r, 1e-12)
        for kb in range(8):
            obuf[j, pl.ds(kb * 16, 16)] = vecs[kb] / denom

    pltpu.sync_copy(obuf, out_hbm.at[pl.ds(wid * 8, 8)])


@functools.cache
def _make_sc_gather():
    return pl.kernel(
        _sc_gather_body,
        out_type=jax.ShapeDtypeStruct((_PAD_SLOTS, _D), jnp.float32),
        mesh=plsc.VectorSubcoreMesh(core_axis_name="c", subcore_axis_name="s"),
        compiler_params=pltpu.CompilerParams(needs_layout_passes=False),
        scratch_types=[
            pltpu.VMEM((8, _D), jnp.int32),
            pltpu.VMEM((8, 16), jnp.int32),
            pltpu.VMEM((_D, _W), jnp.float32),
            pltpu.VMEM((8, _D), jnp.float32),
            pltpu.VMEM((32,), jnp.float32),
            pltpu.SemaphoreType.DMA,
        ],
    )


def kernel(logits, raw_descriptors):
    prob = jax.nn.sigmoid(logits[:, 0])
    vals, idx = _nms_topk(prob)
    vals, idx = vals[:, 0], idx[:, 0]
    y = (idx // _W).astype(jnp.float32)
    x = (idx % _W).astype(jnp.float32)
    positions = jnp.stack([y, x, vals], axis=-1)

    yi = idx // _W
    xi = idx % _W
    base = yi + jnp.array([[0], [_D * _H]], jnp.int32)
    base = jnp.pad(base.reshape(-1), (0, _PAD_SLOTS - _B * _K))
    row_idx = base[:, None] + jnp.arange(_D, dtype=jnp.int32)[None, :] * _H
    xcol = jnp.pad(xi.reshape(-1), (0, _PAD_SLOTS - _B * _K))
    xcol_b = jnp.broadcast_to(xcol[:, None], (_PAD_SLOTS, 16))
    gathered = _make_sc_gather()(raw_descriptors, row_idx, xcol_b)
    sparse_desc = gathered[: _B * _K].reshape(_B, _K, _D)
    return positions, sparse_desc


# speedup vs baseline: 1.2082x; 1.1116x over previous
import functools

import jax
import jax.numpy as jnp
from jax import lax
from jax.experimental import pallas as pl
from jax.experimental.pallas import tpu as pltpu
from jax.experimental.pallas import tpu_sc as plsc

_B, _D, _H, _W = 2, 128, 512, 512
_K = 100
_THR = 0.8
_BORD = 4
_HW = _H * _W
_PAD_SLOTS = 256
_IDX_PAD = 264


def _shift_max_1d(p, axis, dist):
    n = p.shape[axis]
    if axis == 1:
        zero = jnp.zeros((p.shape[0], dist), p.dtype)
        left = jnp.concatenate([p[:, dist:], zero], axis=1)
        right = jnp.concatenate([zero, p[:, : n - dist]], axis=1)
    else:
        zero = jnp.zeros((dist, p.shape[1]), p.dtype)
        left = jnp.concatenate([p[dist:, :], zero], axis=0)
        right = jnp.concatenate([zero, p[: n - dist, :]], axis=0)
    return jnp.maximum(jnp.maximum(left, right), p)


def _nms_topk_body(prob_ref, vals_ref, idx_ref, masked_ref):
    ys = lax.broadcasted_iota(jnp.int32, (_H, _W), 0)
    xs = lax.broadcasted_iota(jnp.int32, (_H, _W), 1)
    border = (ys >= _BORD) & (ys < _H - _BORD) & (xs >= _BORD) & (xs < _W - _BORD)
    rowid = (lax.broadcasted_iota(jnp.int32, (8, _H // 8), 0) * (_H // 8)
             + lax.broadcasted_iota(jnp.int32, (8, _H // 8), 1))
    colid = lax.broadcasted_iota(jnp.int32, (1, _W), 1)
    big = jnp.int32(1 << 20)

    rmax0 = []
    for b in range(_B):
        p = prob_ref[b]
        m = _shift_max_1d(p, 1, 1)
        m = _shift_max_1d(m, 1, 3)
        m = _shift_max_1d(m, 0, 1)
        m = _shift_max_1d(m, 0, 3)
        mask = (p >= m) & (p > _THR) & border
        masked = jnp.where(mask, p, 0.0)
        masked_ref[b] = masked
        rmax0.append(jnp.max(masked.reshape(8, _H // 8, _W), axis=2))

    def step(k, rmaxs):
        out = []
        for b in range(_B):
            rmax = rmaxs[b]
            mval = jnp.max(rmax)
            rid = jnp.min(jnp.where(rmax == mval, rowid, big))
            row = masked_ref[b, pl.ds(rid, 1), :]
            col = jnp.min(jnp.where(row == mval, colid, big))
            vals_ref[b, 0, k] = mval
            idx_ref[b, 0, k] = rid * _W + col
            newrow = jnp.where(colid == col, -1.0, row)
            masked_ref[b, pl.ds(rid, 1), :] = newrow
            out.append(jnp.where(rowid == rid, jnp.max(newrow), rmax))
        return tuple(out)

    lax.fori_loop(0, _K, step, tuple(rmax0))


def _nms_topk(prob):
    return pl.pallas_call(
        _nms_topk_body,
        out_specs=[
            pl.BlockSpec(memory_space=pltpu.SMEM),
            pl.BlockSpec(memory_space=pltpu.SMEM),
        ],
        out_shape=[
            jax.ShapeDtypeStruct((_B, 1, _K), jnp.float32),
            jax.ShapeDtypeStruct((_B, 1, _K), jnp.int32),
        ],
        scratch_shapes=[pltpu.VMEM((_B, _H, _W), jnp.float32)],
    )(prob)


def _sc_gather_body(desc_hbm, idx_hbm, xcol_hbm, out_hbm, ibuf, xwin, rbuf,
                    obuf, tmp, sem):
    wid = lax.axis_index("s") * 2 + lax.axis_index("c")
    @pl.when(wid < (_B * _K + 7) // 8)
    def _work():
        pltpu.sync_copy(idx_hbm.at[pl.ds(wid * 8, 8)], ibuf)
        pltpu.sync_copy(xcol_hbm.at[pl.ds(wid * 8, 8)], xwin)
        table = desc_hbm.reshape(_B * _D * _H, _W)

        lanes16 = lax.broadcasted_iota(jnp.int32, (16,), 0)
        for j in range(8):
            pltpu.async_copy(table.at[ibuf.at[j]], rbuf, sem).wait()
            xv = xwin[j]
            vecs = [
                plsc.load_gather(rbuf, [kb * 16 + lanes16, xv])
                for kb in range(8)
            ]
            acc = vecs[0] * vecs[0]
            for kb in range(1, 8):
                acc = acc + vecs[kb] * vecs[kb]
            s_v = acc
            for off in (8, 4, 2, 1):
                tmp[pl.ds(0, 16)] = s_v
                tmp[pl.ds(16, 16)] = s_v
                s_v = s_v + tmp[pl.ds(off, 16)]
            bits = plsc.bitcast(s_v, jnp.int32)
            r = plsc.bitcast(jnp.int32(0x5F3759DF) - (bits >> 1), jnp.float32)
            for _ in range(3):
                r = r * (1.5 - 0.5 * s_v * r * r)
            denom = jnp.maximum(s_v * r, 1e-12)
            for kb in range(8):
                obuf[j, pl.ds(kb * 16, 16)] = vecs[kb] / denom

        pltpu.sync_copy(obuf, out_hbm.at[pl.ds(wid * 8, 8)])


@functools.cache
def _make_sc_gather():
    return pl.kernel(
        _sc_gather_body,
        out_type=jax.ShapeDtypeStruct((_PAD_SLOTS, _D), jnp.float32),
        mesh=plsc.VectorSubcoreMesh(core_axis_name="c", subcore_axis_name="s"),
        compiler_params=pltpu.CompilerParams(needs_layout_passes=False),
        scratch_types=[
            pltpu.VMEM((8, _D), jnp.int32),
            pltpu.VMEM((8, 16), jnp.int32),
            pltpu.VMEM((_D, _W), jnp.float32),
            pltpu.VMEM((8, _D), jnp.float32),
            pltpu.VMEM((32,), jnp.float32),
            pltpu.SemaphoreType.DMA,
        ],
    )


def kernel(logits, raw_descriptors):
    prob = jax.nn.sigmoid(logits[:, 0])
    vals, idx = _nms_topk(prob)
    vals, idx = vals[:, 0], idx[:, 0]
    y = (idx // _W).astype(jnp.float32)
    x = (idx % _W).astype(jnp.float32)
    positions = jnp.stack([y, x, vals], axis=-1)

    yi = idx // _W
    xi = idx % _W
    base = yi + jnp.array([[0], [_D * _H]], jnp.int32)
    base = jnp.pad(base.reshape(-1), (0, _PAD_SLOTS - _B * _K))
    row_idx = base[:, None] + jnp.arange(_D, dtype=jnp.int32)[None, :] * _H
    xcol = jnp.pad(xi.reshape(-1), (0, _PAD_SLOTS - _B * _K))
    xcol_b = jnp.broadcast_to(xcol[:, None], (_PAD_SLOTS, 16))
    gathered = _make_sc_gather()(raw_descriptors, row_idx, xcol_b)
    sparse_desc = gathered[: _B * _K].reshape(_B, _K, _D)
    return positions, sparse_desc


# speedup vs baseline: 1.2116x; 1.0028x over previous
import functools

import jax
import jax.numpy as jnp
from jax import lax
from jax.experimental import pallas as pl
from jax.experimental.pallas import tpu as pltpu
from jax.experimental.pallas import tpu_sc as plsc

_B, _D, _H, _W = 2, 128, 512, 512
_K = 100
_THR = 0.8
_BORD = 4
_HW = _H * _W
_PAD_SLOTS = 256
_IDX_PAD = 264


def _shift_max_1d(p, axis, dist):
    n = p.shape[axis]
    if axis == 1:
        zero = jnp.zeros((p.shape[0], dist), p.dtype)
        left = jnp.concatenate([p[:, dist:], zero], axis=1)
        right = jnp.concatenate([zero, p[:, : n - dist]], axis=1)
    else:
        zero = jnp.zeros((dist, p.shape[1]), p.dtype)
        left = jnp.concatenate([p[dist:, :], zero], axis=0)
        right = jnp.concatenate([zero, p[: n - dist, :]], axis=0)
    return jnp.maximum(jnp.maximum(left, right), p)


def _nms_topk_body(prob_ref, vals_ref, idx_ref, masked_a, masked_b):
    masked_refs = (masked_a, masked_b)
    ys = lax.broadcasted_iota(jnp.int32, (_H, _W), 0)
    xs = lax.broadcasted_iota(jnp.int32, (_H, _W), 1)
    border = (ys >= _BORD) & (ys < _H - _BORD) & (xs >= _BORD) & (xs < _W - _BORD)
    rowid = (lax.broadcasted_iota(jnp.int32, (8, _H // 8), 0) * (_H // 8)
             + lax.broadcasted_iota(jnp.int32, (8, _H // 8), 1))
    colid = lax.broadcasted_iota(jnp.int32, (1, _W), 1)
    big = jnp.int32(1 << 20)

    rmax0 = []
    for b in range(_B):
        p = prob_ref[b]
        m = _shift_max_1d(p, 1, 1)
        m = _shift_max_1d(m, 1, 3)
        m = _shift_max_1d(m, 0, 1)
        m = _shift_max_1d(m, 0, 3)
        mask = (p >= m) & (p > _THR) & border
        masked = jnp.where(mask, p, 0.0)
        masked_refs[b][...] = masked
        rmax0.append(jnp.max(masked.reshape(8, _H // 8, _W), axis=2))

    def step(k, rmaxs):
        out = []
        for b in range(_B):
            rmax = rmaxs[b]
            mval = jnp.max(rmax)
            rid = jnp.min(jnp.where(rmax == mval, rowid, big))
            row = masked_refs[b][pl.ds(rid, 1), :]
            col = jnp.min(jnp.where(row == mval, colid, big))
            vals_ref[b, 0, k] = mval
            idx_ref[b, 0, k] = rid * _W + col
            newrow = jnp.where(colid == col, -1.0, row)
            masked_refs[b][pl.ds(rid, 1), :] = newrow
            out.append(jnp.where(rowid == rid, jnp.max(newrow), rmax))
        return tuple(out)

    lax.fori_loop(0, _K, step, tuple(rmax0))


def _nms_topk(prob):
    return pl.pallas_call(
        _nms_topk_body,
        out_specs=[
            pl.BlockSpec(memory_space=pltpu.SMEM),
            pl.BlockSpec(memory_space=pltpu.SMEM),
        ],
        out_shape=[
            jax.ShapeDtypeStruct((_B, 1, _K), jnp.float32),
            jax.ShapeDtypeStruct((_B, 1, _K), jnp.int32),
        ],
        scratch_shapes=[pltpu.VMEM((_H, _W), jnp.float32),
                        pltpu.VMEM((_H, _W), jnp.float32)],
    )(prob)


def _sc_gather_body(desc_hbm, idx_hbm, xcol_hbm, out_hbm, ibuf, xwin, rbuf,
                    obuf, tmp, sem):
    wid = lax.axis_index("s") * 2 + lax.axis_index("c")
    @pl.when(wid < (_B * _K + 7) // 8)
    def _work():
        pltpu.sync_copy(idx_hbm.at[pl.ds(wid * 8, 8)], ibuf)
        pltpu.sync_copy(xcol_hbm.at[pl.ds(wid * 8, 8)], xwin)
        table = desc_hbm.reshape(_B * _D * _H, _W)

        lanes16 = lax.broadcasted_iota(jnp.int32, (16,), 0)
        for j in range(8):
            pltpu.async_copy(table.at[ibuf.at[j]], rbuf, sem).wait()
            xv = xwin[j]
            vecs = [
                plsc.load_gather(rbuf, [kb * 16 + lanes16, xv])
                for kb in range(8)
            ]
            acc = vecs[0] * vecs[0]
            for kb in range(1, 8):
                acc = acc + vecs[kb] * vecs[kb]
            s_v = acc
            for off in (8, 4, 2, 1):
                tmp[pl.ds(0, 16)] = s_v
                tmp[pl.ds(16, 16)] = s_v
                s_v = s_v + tmp[pl.ds(off, 16)]
            bits = plsc.bitcast(s_v, jnp.int32)
            r = plsc.bitcast(jnp.int32(0x5F3759DF) - (bits >> 1), jnp.float32)
            for _ in range(3):
                r = r * (1.5 - 0.5 * s_v * r * r)
            denom = jnp.maximum(s_v * r, 1e-12)
            for kb in range(8):
                obuf[j, pl.ds(kb * 16, 16)] = vecs[kb] / denom

        pltpu.sync_copy(obuf, out_hbm.at[pl.ds(wid * 8, 8)])


@functools.cache
def _make_sc_gather():
    return pl.kernel(
        _sc_gather_body,
        out_type=jax.ShapeDtypeStruct((_PAD_SLOTS, _D), jnp.float32),
        mesh=plsc.VectorSubcoreMesh(core_axis_name="c", subcore_axis_name="s"),
        compiler_params=pltpu.CompilerParams(needs_layout_passes=False),
        scratch_types=[
            pltpu.VMEM((8, _D), jnp.int32),
            pltpu.VMEM((8, 16), jnp.int32),
            pltpu.VMEM((_D, _W), jnp.float32),
            pltpu.VMEM((8, _D), jnp.float32),
            pltpu.VMEM((32,), jnp.float32),
            pltpu.SemaphoreType.DMA,
        ],
    )


def kernel(logits, raw_descriptors):
    prob = jax.nn.sigmoid(logits[:, 0])
    vals, idx = _nms_topk(prob)
    vals, idx = vals[:, 0], idx[:, 0]
    y = (idx // _W).astype(jnp.float32)
    x = (idx % _W).astype(jnp.float32)
    positions = jnp.stack([y, x, vals], axis=-1)

    yi = idx // _W
    xi = idx % _W
    base = yi + jnp.array([[0], [_D * _H]], jnp.int32)
    base = jnp.pad(base.reshape(-1), (0, _PAD_SLOTS - _B * _K))
    row_idx = base[:, None] + jnp.arange(_D, dtype=jnp.int32)[None, :] * _H
    xcol = jnp.pad(xi.reshape(-1), (0, _PAD_SLOTS - _B * _K))
    xcol_b = jnp.broadcast_to(xcol[:, None], (_PAD_SLOTS, 16))
    gathered = _make_sc_gather()(raw_descriptors, row_idx, xcol_b)
    sparse_desc = gathered[: _B * _K].reshape(_B, _K, _D)
    return positions, sparse_desc


# speedup vs baseline: 1.2165x; 1.0040x over previous
import functools

import jax
import jax.numpy as jnp
from jax import lax
from jax.experimental import pallas as pl
from jax.experimental.pallas import tpu as pltpu
from jax.experimental.pallas import tpu_sc as plsc

_B, _D, _H, _W = 2, 128, 512, 512
_K = 100
_THR = 0.8
_BORD = 4
_HW = _H * _W
_PAD_SLOTS = 256
_IDX_PAD = 264


def _shift_max_1d(p, axis, dist):
    n = p.shape[axis]
    if axis == 1:
        zero = jnp.zeros((p.shape[0], dist), p.dtype)
        left = jnp.concatenate([p[:, dist:], zero], axis=1)
        right = jnp.concatenate([zero, p[:, : n - dist]], axis=1)
    else:
        zero = jnp.zeros((dist, p.shape[1]), p.dtype)
        left = jnp.concatenate([p[dist:, :], zero], axis=0)
        right = jnp.concatenate([zero, p[: n - dist, :]], axis=0)
    return jnp.maximum(jnp.maximum(left, right), p)


def _nms_topk_body(prob_ref, vals_ref, idx_ref, masked_a, masked_b):
    masked_refs = (masked_a, masked_b)
    ys = lax.broadcasted_iota(jnp.int32, (_H, _W), 0)
    xs = lax.broadcasted_iota(jnp.int32, (_H, _W), 1)
    border = (ys >= _BORD) & (ys < _H - _BORD) & (xs >= _BORD) & (xs < _W - _BORD)
    rowid = (lax.broadcasted_iota(jnp.int32, (8, _H // 8), 0) * (_H // 8)
             + lax.broadcasted_iota(jnp.int32, (8, _H // 8), 1))
    colid = (lax.broadcasted_iota(jnp.int32, (8, _W // 8), 0) * (_W // 8)
             + lax.broadcasted_iota(jnp.int32, (8, _W // 8), 1))
    big = jnp.int32(1 << 20)

    rmax0 = []
    for b in range(_B):
        p = prob_ref[b]
        m = _shift_max_1d(p, 1, 1)
        m = _shift_max_1d(m, 1, 3)
        m = _shift_max_1d(m, 0, 1)
        m = _shift_max_1d(m, 0, 3)
        mask = (p >= m) & (p > _THR) & border
        masked = jnp.where(mask, p, 0.0)
        masked_refs[b][...] = masked.reshape(_H, 8, _W // 8)
        rmax0.append(jnp.max(masked.reshape(8, _H // 8, _W), axis=2))

    def step(k, rmaxs):
        out = []
        for b in range(_B):
            rmax = rmaxs[b]
            mval = jnp.max(rmax)
            rid = jnp.min(jnp.where(rmax == mval, rowid, big))
            row = masked_refs[b][rid]
            col = jnp.min(jnp.where(row == mval, colid, big))
            vals_ref[b, 0, k] = mval
            idx_ref[b, 0, k] = rid * _W + col
            newrow = jnp.where(colid == col, -1.0, row)
            masked_refs[b][rid] = newrow
            out.append(jnp.where(rowid == rid, jnp.max(newrow), rmax))
        return tuple(out)

    lax.fori_loop(0, _K, step, tuple(rmax0))


def _nms_topk(prob):
    return pl.pallas_call(
        _nms_topk_body,
        out_specs=[
            pl.BlockSpec(memory_space=pltpu.SMEM),
            pl.BlockSpec(memory_space=pltpu.SMEM),
        ],
        out_shape=[
            jax.ShapeDtypeStruct((_B, 1, _K), jnp.float32),
            jax.ShapeDtypeStruct((_B, 1, _K), jnp.int32),
        ],
        scratch_shapes=[pltpu.VMEM((_H, 8, _W // 8), jnp.float32),
                        pltpu.VMEM((_H, 8, _W // 8), jnp.float32)],
    )(prob)


def _sc_gather_body(desc_hbm, idx_hbm, xcol_hbm, out_hbm, ibuf, xwin, rbuf,
                    obuf, tmp, sem):
    wid = lax.axis_index("s") * 2 + lax.axis_index("c")
    @pl.when(wid < (_B * _K + 7) // 8)
    def _work():
        pltpu.sync_copy(idx_hbm.at[pl.ds(wid * 8, 8)], ibuf)
        pltpu.sync_copy(xcol_hbm.at[pl.ds(wid * 8, 8)], xwin)
        table = desc_hbm.reshape(_B * _D * _H, _W)

        lanes16 = lax.broadcasted_iota(jnp.int32, (16,), 0)
        for j in range(8):
            pltpu.async_copy(table.at[ibuf.at[j]], rbuf, sem).wait()
            xv = xwin[j]
            vecs = [
                plsc.load_gather(rbuf, [kb * 16 + lanes16, xv])
                for kb in range(8)
            ]
            acc = vecs[0] * vecs[0]
            for kb in range(1, 8):
                acc = acc + vecs[kb] * vecs[kb]
            s_v = acc
            for off in (8, 4, 2, 1):
                tmp[pl.ds(0, 16)] = s_v
                tmp[pl.ds(16, 16)] = s_v
                s_v = s_v + tmp[pl.ds(off, 16)]
            bits = plsc.bitcast(s_v, jnp.int32)
            r = plsc.bitcast(jnp.int32(0x5F3759DF) - (bits >> 1), jnp.float32)
            for _ in range(3):
                r = r * (1.5 - 0.5 * s_v * r * r)
            denom = jnp.maximum(s_v * r, 1e-12)
            for kb in range(8):
                obuf[j, pl.ds(kb * 16, 16)] = vecs[kb] / denom

        pltpu.sync_copy(obuf, out_hbm.at[pl.ds(wid * 8, 8)])


@functools.cache
def _make_sc_gather():
    return pl.kernel(
        _sc_gather_body,
        out_type=jax.ShapeDtypeStruct((_PAD_SLOTS, _D), jnp.float32),
        mesh=plsc.VectorSubcoreMesh(core_axis_name="c", subcore_axis_name="s"),
        compiler_params=pltpu.CompilerParams(needs_layout_passes=False),
        scratch_types=[
            pltpu.VMEM((8, _D), jnp.int32),
            pltpu.VMEM((8, 16), jnp.int32),
            pltpu.VMEM((_D, _W), jnp.float32),
            pltpu.VMEM((8, _D), jnp.float32),
            pltpu.VMEM((32,), jnp.float32),
            pltpu.SemaphoreType.DMA,
        ],
    )


def kernel(logits, raw_descriptors):
    prob = jax.nn.sigmoid(logits[:, 0])
    vals, idx = _nms_topk(prob)
    vals, idx = vals[:, 0], idx[:, 0]
    y = (idx // _W).astype(jnp.float32)
    x = (idx % _W).astype(jnp.float32)
    positions = jnp.stack([y, x, vals], axis=-1)

    yi = idx // _W
    xi = idx % _W
    base = yi + jnp.array([[0], [_D * _H]], jnp.int32)
    base = jnp.pad(base.reshape(-1), (0, _PAD_SLOTS - _B * _K))
    row_idx = base[:, None] + jnp.arange(_D, dtype=jnp.int32)[None, :] * _H
    xcol = jnp.pad(xi.reshape(-1), (0, _PAD_SLOTS - _B * _K))
    xcol_b = jnp.broadcast_to(xcol[:, None], (_PAD_SLOTS, 16))
    gathered = _make_sc_gather()(raw_descriptors, row_idx, xcol_b)
    sparse_desc = gathered[: _B * _K].reshape(_B, _K, _D)
    return positions, sparse_desc
